# Initial kernel scaffold; baseline (speedup 1.0000x reference)
import jax
import jax.numpy as jnp
from jax.experimental import pallas as pl


def kernel(inputs, unq_inv):
    raise NotImplementedError("write your pallas kernel here")


# speedup vs baseline: 2.7008x; 2.7008x over previous
import functools

import jax
import jax.numpy as jnp
from jax import lax
from jax.experimental import pallas as pl
from jax.experimental.pallas import tpu as pltpu
from jax.experimental.pallas import tpu_sc as plsc

N_ROWS = 320000
N_FEAT = 128
N_SEG = 10000
N_CORES = 2
N_SUBCORES = 16
ROWS_PER_TILE = N_ROWS // N_SUBCORES
CHUNK = 80
N_CHUNKS = ROWS_PER_TILE // CHUNK
N_SEG_PAD = 10240
SEG_PER_TILE = N_SEG_PAD // N_SUBCORES


def _sc_body(x_hbm, ids_hbm, zs_hbm, on_hbm, out_s, out_c,
             idx_v, rows_v, acc_sh):
    c = lax.axis_index("c")
    s = lax.axis_index("s")
    seg0 = s * SEG_PER_TILE

    pltpu.sync_copy(zs_hbm, acc_sh.at[pl.ds(seg0, SEG_PER_TILE)])
    pltpu.sync_copy(on_hbm, rows_v)
    plsc.subcore_barrier()

    def step(i, _):
        base = s * ROWS_PER_TILE + i * CHUNK
        pltpu.sync_copy(ids_hbm.at[pl.ds(base, CHUNK)], idx_v)

        @pl.when(c == 0)
        def _():
            pltpu.sync_copy(x_hbm.at[pl.ds(base, CHUNK)], rows_v)

        pltpu.sync_copy(rows_v, acc_sh.at[idx_v], add=True)
        return 0

    lax.fori_loop(0, N_CHUNKS, step, 0)

    plsc.subcore_barrier()

    @pl.when(c == 0)
    def _():
        pltpu.sync_copy(acc_sh.at[pl.ds(seg0, SEG_PER_TILE)],
                        out_s.at[pl.ds(seg0, SEG_PER_TILE)])

    @pl.when(c == 1)
    def _():
        pltpu.sync_copy(acc_sh.at[pl.ds(seg0, SEG_PER_TILE)],
                        out_c.at[pl.ds(seg0, SEG_PER_TILE)])


_sc_accumulate = functools.partial(
    pl.kernel,
    out_type=(
        jax.ShapeDtypeStruct((N_SEG_PAD, N_FEAT), jnp.float32),
        jax.ShapeDtypeStruct((N_SEG_PAD, N_FEAT), jnp.float32),
    ),
    mesh=plsc.VectorSubcoreMesh(core_axis_name="c", subcore_axis_name="s"),
    scratch_types=(
        pltpu.VMEM((CHUNK,), jnp.int32),
        pltpu.VMEM((CHUNK, N_FEAT), jnp.float32),
        pltpu.VMEM_SHARED((N_SEG_PAD, N_FEAT), jnp.float32),
    ),
)(_sc_body)


def _combine_body(ps_ref, pc_ref, o_ref):
    o_ref[...] = ps_ref[...] / jnp.maximum(pc_ref[:, 0:1], 1.0)


_combine = pl.pallas_call(
    _combine_body,
    grid=(10,),
    in_specs=[
        pl.BlockSpec((1000, N_FEAT), lambda j: (j, 0)),
        pl.BlockSpec((1000, N_FEAT), lambda j: (j, 0)),
    ],
    out_specs=pl.BlockSpec((1000, N_FEAT), lambda j: (j, 0)),
    out_shape=jax.ShapeDtypeStruct((N_SEG, N_FEAT), jnp.float32),
)


@jax.jit
def kernel(inputs, unq_inv):
    ids = unq_inv.astype(jnp.int32)
    zs = jnp.zeros((SEG_PER_TILE, N_FEAT), jnp.float32)
    on = jnp.ones((CHUNK, N_FEAT), jnp.float32)
    sums, cnts = _sc_accumulate(inputs, ids, zs, on)
    return _combine(sums, cnts)


# speedup vs baseline: 5.3427x; 1.9782x over previous
import functools

import jax
import jax.numpy as jnp
from jax import lax
from jax.experimental import pallas as pl
from jax.experimental.pallas import tpu as pltpu
from jax.experimental.pallas import tpu_sc as plsc

N_ROWS = 320000
N_FEAT = 128
N_SEG = 10000
N_CORES = 2
N_SUBCORES = 16
ROWS_PER_TILE = N_ROWS // N_SUBCORES
CHUNK = 128
N_CHUNKS = ROWS_PER_TILE // CHUNK
REM = ROWS_PER_TILE - N_CHUNKS * CHUNK
N_SEG_PAD = 10240
SEG_PER_TILE = N_SEG_PAD // N_SUBCORES


def _sc_body(x_hbm, ids_hbm, zs_hbm, on_hbm, out_s, out_c,
             idx_a, idx_b, rows_a, rows_b, idx_r, rows_r,
             sem_ia, sem_ib, sem_ra, sem_rb, acc_sh):
    c = lax.axis_index("c")
    s = lax.axis_index("s")
    seg0 = s * SEG_PER_TILE
    row_base = s * ROWS_PER_TILE
    idx_v = (idx_a, idx_b)
    rows_v = (rows_a, rows_b)
    sem_i = (sem_ia, sem_ib)
    sem_r = (sem_ra, sem_rb)

    pltpu.sync_copy(zs_hbm, acc_sh.at[pl.ds(seg0, SEG_PER_TILE)])
    pltpu.sync_copy(on_hbm, rows_a)
    pltpu.sync_copy(on_hbm, rows_b)
    pltpu.sync_copy(on_hbm.at[pl.ds(0, REM)], rows_r)
    plsc.subcore_barrier()

    def issue(i, b):
        base = row_base + i * CHUNK
        pltpu.async_copy(ids_hbm.at[pl.ds(base, CHUNK)], idx_v[b], sem_i[b])

        @pl.when(c == 0)
        def _():
            pltpu.async_copy(x_hbm.at[pl.ds(base, CHUNK)], rows_v[b], sem_r[b])

    def wait_slot(b):
        pltpu.make_async_copy(ids_hbm.at[pl.ds(0, CHUNK)], idx_v[b],
                              sem_i[b]).wait()

        @pl.when(c == 0)
        def _():
            pltpu.make_async_copy(x_hbm.at[pl.ds(0, CHUNK)], rows_v[b],
                                  sem_r[b]).wait()

    issue(0, 0)

    def step(g, _):
        for b in range(2):
            i = g * 2 + b
            wait_slot(b)

            @pl.when(i + 1 < N_CHUNKS)
            def _():
                issue(i + 1, b ^ 1)

            pltpu.sync_copy(rows_v[b], acc_sh.at[idx_v[b]], add=True)
        return 0

    lax.fori_loop(0, N_CHUNKS // 2, step, 0)

    rem_base = row_base + N_CHUNKS * CHUNK
    pltpu.sync_copy(ids_hbm.at[pl.ds(rem_base, REM)], idx_r)

    @pl.when(c == 0)
    def _():
        pltpu.sync_copy(x_hbm.at[pl.ds(rem_base, REM)], rows_r)

    pltpu.sync_copy(rows_r, acc_sh.at[idx_r], add=True)

    plsc.subcore_barrier()

    @pl.when(c == 0)
    def _():
        pltpu.sync_copy(acc_sh.at[pl.ds(seg0, SEG_PER_TILE)],
                        out_s.at[pl.ds(seg0, SEG_PER_TILE)])

    @pl.when(c == 1)
    def _():
        pltpu.sync_copy(acc_sh.at[pl.ds(seg0, SEG_PER_TILE)],
                        out_c.at[pl.ds(seg0, SEG_PER_TILE)])


_sc_accumulate = functools.partial(
    pl.kernel,
    out_type=(
        jax.ShapeDtypeStruct((N_SEG_PAD, N_FEAT), jnp.float32),
        jax.ShapeDtypeStruct((N_SEG_PAD, N_FEAT), jnp.float32),
    ),
    mesh=plsc.VectorSubcoreMesh(core_axis_name="c", subcore_axis_name="s"),
    scratch_types=(
        pltpu.VMEM((CHUNK,), jnp.int32),
        pltpu.VMEM((CHUNK,), jnp.int32),
        pltpu.VMEM((CHUNK, N_FEAT), jnp.float32),
        pltpu.VMEM((CHUNK, N_FEAT), jnp.float32),
        pltpu.VMEM((REM,), jnp.int32),
        pltpu.VMEM((REM, N_FEAT), jnp.float32),
        pltpu.SemaphoreType.DMA,
        pltpu.SemaphoreType.DMA,
        pltpu.SemaphoreType.DMA,
        pltpu.SemaphoreType.DMA,
        pltpu.VMEM_SHARED((N_SEG_PAD, N_FEAT), jnp.float32),
    ),
)(_sc_body)


def _combine_body(ps_ref, pc_ref, o_ref):
    o_ref[...] = ps_ref[...] / jnp.maximum(pc_ref[:, 0:1], 1.0)


_combine = pl.pallas_call(
    _combine_body,
    grid=(10,),
    in_specs=[
        pl.BlockSpec((1000, N_FEAT), lambda j: (j, 0)),
        pl.BlockSpec((1000, N_FEAT), lambda j: (j, 0)),
    ],
    out_specs=pl.BlockSpec((1000, N_FEAT), lambda j: (j, 0)),
    out_shape=jax.ShapeDtypeStruct((N_SEG, N_FEAT), jnp.float32),
)


@jax.jit
def kernel(inputs, unq_inv):
    ids = unq_inv.astype(jnp.int32)
    zs = jnp.zeros((SEG_PER_TILE, N_FEAT), jnp.float32)
    on = jnp.ones((CHUNK, N_FEAT), jnp.float32)
    sums, cnts = _sc_accumulate(inputs, ids, zs, on)
    return _combine(sums, cnts)
